# Initial kernel scaffold; baseline (speedup 1.0000x reference)
import jax
import jax.numpy as jnp
from jax.experimental import pallas as pl


def kernel(pos_u, pos_v, neg_v, U, V):
    raise NotImplementedError("write your pallas kernel here")


# speedup vs baseline: 3.7636x; 3.7636x over previous
import jax
import jax.numpy as jnp
from jax import lax
from jax.experimental import pallas as pl
from jax.experimental.pallas import tpu as pltpu
from jax.experimental.pallas import tpu_sc as plsc

B = 16384
D = 64
N_NEG = 20
LANES = 16
NW = 32
PER_W = B // NW
C = 32
NCHUNK = PER_W // C
NEG_ROWS = C * N_NEG
IDX_W = 128
NEG_GATHERS = NEG_ROWS // IDX_W
VOCAB2 = 500000


def _split_idx(raw):
    return raw >> 1, (raw & 1) << 6


def _sc_body(pos_u_hbm, pos_v_hbm, neg_v_hbm, u_hbm, v_hbm,
             pos_out_hbm, neg_out_hbm,
             idx_u_raw, idx_u_row, idx_u_off,
             idx_pv_raw, idx_pv_row, idx_pv_off,
             idx_nv_raw, idx_nv_row, idx_nv_off,
             u_rows, pv_rows, nv_rows,
             out_pos, out_neg, sem_u, sem_pv, sem_nv):
    wid = lax.axis_index("s") * 2 + lax.axis_index("c")
    lane = lax.iota(jnp.int32, LANES)

    def chunk(c, carry):
        base = wid * PER_W + c * C
        pltpu.sync_copy(pos_u_hbm.at[pl.ds(base, C)], idx_u_raw)
        pltpu.sync_copy(pos_v_hbm.at[pl.ds(base, C)], idx_pv_raw)
        pltpu.sync_copy(neg_v_hbm.at[pl.ds(base * N_NEG, NEG_ROWS)],
                        idx_nv_raw)

        for j in range(C // LANES):
            sl = pl.ds(j * LANES, LANES)
            row, off = _split_idx(idx_u_raw[sl])
            idx_u_row[sl], idx_u_off[sl] = row, off
            row, off = _split_idx(idx_pv_raw[sl])
            idx_pv_row[sl], idx_pv_off[sl] = row, off

        def tr(j, carry2):
            sl = pl.ds(j * LANES, LANES)
            row, off = _split_idx(idx_nv_raw[sl])
            idx_nv_row[sl], idx_nv_off[sl] = row, off
            return carry2

        lax.fori_loop(0, NEG_ROWS // LANES, tr, 0)

        cps = [pltpu.async_copy(u_hbm.at[idx_u_row], u_rows, sem_u),
               pltpu.async_copy(v_hbm.at[idx_pv_row], pv_rows, sem_pv)]
        for j in range(NEG_GATHERS):
            sl = pl.ds(j * IDX_W, IDX_W)
            cps.append(pltpu.async_copy(v_hbm.at[idx_nv_row.at[sl]],
                                        nv_rows.at[sl], sem_nv))
        for cp in cps:
            cp.wait()

        def group(g, carry2):
            i_vec = g * LANES + lane
            gsl = pl.ds(g * LANES, LANES)
            off_u = idx_u_off[gsl]
            off_p = idx_pv_off[gsl]
            i20 = i_vec * N_NEG
            zero = jnp.zeros((LANES,), jnp.float32)

            rows_a = [i20 + n for n in range(N_NEG // 2)]
            offs_a = [plsc.load_gather(idx_nv_off, [r]) for r in rows_a]

            def dstep_a(d, accs):
                u_val = plsc.load_gather(u_rows, [i_vec, off_u + d])
                p_val = plsc.load_gather(pv_rows, [i_vec, off_p + d])
                new = [accs[0] + p_val * u_val]
                for k in range(N_NEG // 2):
                    v = plsc.load_gather(nv_rows, [rows_a[k], offs_a[k] + d])
                    new.append(accs[k + 1] + v * u_val)
                return tuple(new)

            accs = lax.fori_loop(0, D, dstep_a, (zero,) * (N_NEG // 2 + 1))
            out_pos[gsl] = accs[0]
            for k in range(N_NEG // 2):
                plsc.store_scatter(out_neg, [rows_a[k]], accs[k + 1])

            rows_b = [i20 + n for n in range(N_NEG // 2, N_NEG)]
            offs_b = [plsc.load_gather(idx_nv_off, [r]) for r in rows_b]

            def dstep_b(d, accs):
                u_val = plsc.load_gather(u_rows, [i_vec, off_u + d])
                new = []
                for k in range(N_NEG // 2):
                    v = plsc.load_gather(nv_rows, [rows_b[k], offs_b[k] + d])
                    new.append(accs[k] + v * u_val)
                return tuple(new)

            accs = lax.fori_loop(0, D, dstep_b, (zero,) * (N_NEG // 2))
            for k in range(N_NEG // 2):
                plsc.store_scatter(out_neg, [rows_b[k]], accs[k])
            return carry2

        lax.fori_loop(0, C // LANES, group, 0)
        pltpu.sync_copy(out_pos, pos_out_hbm.at[pl.ds(base, C)])
        pltpu.sync_copy(out_neg, neg_out_hbm.at[pl.ds(base * N_NEG, NEG_ROWS)])
        return carry

    lax.fori_loop(0, NCHUNK, chunk, 0)


def kernel(pos_u, pos_v, neg_v, U, V):
    pos_u_i = pos_u.astype(jnp.int32)
    pos_v_i = pos_v.astype(jnp.int32).reshape(B)
    neg_v_i = neg_v.astype(jnp.int32).reshape(B * N_NEG)
    u128 = U.reshape(VOCAB2, 2 * D)
    v128 = V.reshape(VOCAB2, 2 * D)

    mesh = plsc.VectorSubcoreMesh(core_axis_name="c", subcore_axis_name="s")
    run = pl.kernel(
        _sc_body,
        out_type=(jax.ShapeDtypeStruct((B,), jnp.float32),
                  jax.ShapeDtypeStruct((B * N_NEG,), jnp.float32)),
        mesh=mesh,
        compiler_params=pltpu.CompilerParams(needs_layout_passes=False),
        scratch_types=[
            pltpu.VMEM((C,), jnp.int32),
            pltpu.VMEM((C,), jnp.int32),
            pltpu.VMEM((C,), jnp.int32),
            pltpu.VMEM((C,), jnp.int32),
            pltpu.VMEM((C,), jnp.int32),
            pltpu.VMEM((C,), jnp.int32),
            pltpu.VMEM((NEG_ROWS,), jnp.int32),
            pltpu.VMEM((NEG_ROWS,), jnp.int32),
            pltpu.VMEM((NEG_ROWS,), jnp.int32),
            pltpu.VMEM((C, 2 * D), jnp.float32),
            pltpu.VMEM((C, 2 * D), jnp.float32),
            pltpu.VMEM((NEG_ROWS, 2 * D), jnp.float32),
            pltpu.VMEM((C,), jnp.float32),
            pltpu.VMEM((NEG_ROWS,), jnp.float32),
            pltpu.SemaphoreType.DMA,
            pltpu.SemaphoreType.DMA,
            pltpu.SemaphoreType.DMA,
        ],
    )
    pos_flat, neg_flat = run(pos_u_i, pos_v_i, neg_v_i, u128, v128)
    return (pos_flat.reshape(B, 1), neg_flat.reshape(B, N_NEG))


# speedup vs baseline: 4.5555x; 1.2104x over previous
import jax
import jax.numpy as jnp
from jax import lax
from jax.experimental import pallas as pl
from jax.experimental.pallas import tpu as pltpu
from jax.experimental.pallas import tpu_sc as plsc

B = 16384
D = 64
N_NEG = 20
LANES = 16
NW = 32
PER_W = B // NW
C = 32
NCHUNK = PER_W // C
NEG_ROWS = C * N_NEG
IDX_W = 128
NEG_GATHERS = NEG_ROWS // IDX_W
VOCAB2 = 500000


def _sc_body(pos_u_hbm, pos_v_hbm, neg_v_hbm, u_hbm, v_hbm,
             pos_out_hbm, neg_out_hbm,
             idx_u_raw, idx_u_row, idx_pv_raw, idx_pv_row,
             idx_nv_raw, idx_nv_row,
             off_u_s, off_pv_s, off_nv_s,
             u_rows, pv_rows, nv_rows,
             out_pos, out_neg, sem_u, sem_pv, sem_nv):
    wid = lax.axis_index("s") * 2 + lax.axis_index("c")
    lane = lax.iota(jnp.int32, LANES)

    def chunk(c, carry):
        base = wid * PER_W + c * C
        pltpu.sync_copy(pos_u_hbm.at[pl.ds(base, C)], idx_u_raw)
        pltpu.sync_copy(pos_v_hbm.at[pl.ds(base, C)], idx_pv_raw)
        pltpu.sync_copy(neg_v_hbm.at[pl.ds(base * N_NEG, NEG_ROWS)],
                        idx_nv_raw)

        for j in range(C // LANES):
            sl = pl.ds(j * LANES, LANES)
            vu = idx_u_raw[sl]
            idx_u_row[sl] = vu >> 1
            vp = idx_pv_raw[sl]
            idx_pv_row[sl] = vp >> 1
            for l in range(LANES):
                off_u_s[j * LANES + l] = (vu[l] & 1) << 6
                off_pv_s[j * LANES + l] = (vp[l] & 1) << 6

        def tr(j, carry2):
            sl = pl.ds(j * LANES, LANES)
            vn = idx_nv_raw[sl]
            idx_nv_row[sl] = vn >> 1
            for l in range(LANES):
                off_nv_s[j * LANES + l] = (vn[l] & 1) << 6
            return carry2

        lax.fori_loop(0, NEG_ROWS // LANES, tr, 0)

        cps = [pltpu.async_copy(u_hbm.at[idx_u_row], u_rows, sem_u),
               pltpu.async_copy(v_hbm.at[idx_pv_row], pv_rows, sem_pv)]
        for j in range(NEG_GATHERS):
            sl = pl.ds(j * IDX_W, IDX_W)
            cps.append(pltpu.async_copy(v_hbm.at[idx_nv_row.at[sl]],
                                        nv_rows.at[sl], sem_nv))
        for cp in cps:
            cp.wait()

        m_last = lane == (LANES - 1)

        def ex(i, carry2):
            off_u = off_u_s[i]
            u0 = u_rows[i, pl.ds(off_u, 16)]
            u1 = u_rows[i, pl.ds(off_u + 16, 16)]
            u2 = u_rows[i, pl.ds(off_u + 32, 16)]
            u3 = u_rows[i, pl.ds(off_u + 48, 16)]
            off_p = off_pv_s[i]
            acc = (pv_rows[i, pl.ds(off_p, 16)] * u0
                   + pv_rows[i, pl.ds(off_p + 16, 16)] * u1
                   + pv_rows[i, pl.ds(off_p + 32, 16)] * u2
                   + pv_rows[i, pl.ds(off_p + 48, 16)] * u3)
            s = plsc.cumsum(acc)
            plsc.store_scatter(out_pos, [jnp.full((LANES,), i, jnp.int32)],
                               s, mask=m_last)

            def ng(n, carry3):
                r = i * N_NEG + n
                off = off_nv_s[r]
                accn = (nv_rows[r, pl.ds(off, 16)] * u0
                        + nv_rows[r, pl.ds(off + 16, 16)] * u1
                        + nv_rows[r, pl.ds(off + 32, 16)] * u2
                        + nv_rows[r, pl.ds(off + 48, 16)] * u3)
                sn = plsc.cumsum(accn)
                plsc.store_scatter(out_neg,
                                   [jnp.full((LANES,), r, jnp.int32)],
                                   sn, mask=m_last)
                return carry3

            lax.fori_loop(0, N_NEG, ng, 0)
            return carry2

        lax.fori_loop(0, C, ex, 0)
        pltpu.sync_copy(out_pos, pos_out_hbm.at[pl.ds(base, C)])
        pltpu.sync_copy(out_neg, neg_out_hbm.at[pl.ds(base * N_NEG, NEG_ROWS)])
        return carry

    lax.fori_loop(0, NCHUNK, chunk, 0)


def kernel(pos_u, pos_v, neg_v, U, V):
    pos_u_i = pos_u.astype(jnp.int32)
    pos_v_i = pos_v.astype(jnp.int32).reshape(B)
    neg_v_i = neg_v.astype(jnp.int32).reshape(B * N_NEG)
    u128 = U.reshape(VOCAB2, 2 * D)
    v128 = V.reshape(VOCAB2, 2 * D)

    mesh = plsc.VectorSubcoreMesh(core_axis_name="c", subcore_axis_name="s")
    run = pl.kernel(
        _sc_body,
        out_type=(jax.ShapeDtypeStruct((B,), jnp.float32),
                  jax.ShapeDtypeStruct((B * N_NEG,), jnp.float32)),
        mesh=mesh,
        compiler_params=pltpu.CompilerParams(needs_layout_passes=False),
        scratch_types=[
            pltpu.VMEM((C,), jnp.int32),
            pltpu.VMEM((C,), jnp.int32),
            pltpu.VMEM((C,), jnp.int32),
            pltpu.VMEM((C,), jnp.int32),
            pltpu.VMEM((NEG_ROWS,), jnp.int32),
            pltpu.VMEM((NEG_ROWS,), jnp.int32),
            pltpu.SMEM((C,), jnp.int32),
            pltpu.SMEM((C,), jnp.int32),
            pltpu.SMEM((NEG_ROWS,), jnp.int32),
            pltpu.VMEM((C, 2 * D), jnp.float32),
            pltpu.VMEM((C, 2 * D), jnp.float32),
            pltpu.VMEM((NEG_ROWS, 2 * D), jnp.float32),
            pltpu.VMEM((C,), jnp.float32),
            pltpu.VMEM((NEG_ROWS,), jnp.float32),
            pltpu.SemaphoreType.DMA,
            pltpu.SemaphoreType.DMA,
            pltpu.SemaphoreType.DMA,
        ],
    )
    pos_flat, neg_flat = run(pos_u_i, pos_v_i, neg_v_i, u128, v128)
    return (pos_flat.reshape(B, 1), neg_flat.reshape(B, N_NEG))


# speedup vs baseline: 4.7084x; 1.0336x over previous
import jax
import jax.numpy as jnp
from jax import lax
from jax.experimental import pallas as pl
from jax.experimental.pallas import tpu as pltpu
from jax.experimental.pallas import tpu_sc as plsc

B = 16384
D = 64
N_NEG = 20
LANES = 16
NW = 32
PER_W = B // NW
C = 32
NCHUNK = PER_W // C
NEG_ROWS = C * N_NEG
IDX_W = 128
NEG_GATHERS = NEG_ROWS // IDX_W
ROW_W = 64


def _sc_body(pos_u_hbm, pos_v_hbm, neg_v_hbm, u_hbm, v_hbm,
             pos_out_hbm, neg_out_hbm,
             idx_u, idx_pv, idx_nv,
             u_rows, pv_rows, nv_rows,
             out_pos, out_neg, sem_u, sem_pv, sem_nv):
    wid = lax.axis_index("s") * 2 + lax.axis_index("c")
    lane = lax.iota(jnp.int32, LANES)
    m_last = lane == (LANES - 1)

    def chunk(c, carry):
        base = wid * PER_W + c * C
        pltpu.sync_copy(pos_u_hbm.at[pl.ds(base, C)], idx_u)
        pltpu.sync_copy(pos_v_hbm.at[pl.ds(base, C)], idx_pv)
        pltpu.sync_copy(neg_v_hbm.at[pl.ds(base * N_NEG, NEG_ROWS)], idx_nv)

        cps = [pltpu.async_copy(u_hbm.at[idx_u], u_rows, sem_u),
               pltpu.async_copy(v_hbm.at[idx_pv], pv_rows, sem_pv)]
        for j in range(NEG_GATHERS):
            sl = pl.ds(j * IDX_W, IDX_W)
            cps.append(pltpu.async_copy(v_hbm.at[idx_nv.at[sl]],
                                        nv_rows.at[sl], sem_nv))
        for cp in cps:
            cp.wait()

        def ex(i, carry2):
            u0 = u_rows[i, pl.ds(0, 16)]
            u1 = u_rows[i, pl.ds(16, 16)]
            u2 = u_rows[i, pl.ds(32, 16)]
            u3 = u_rows[i, pl.ds(48, 16)]
            acc = (pv_rows[i, pl.ds(0, 16)] * u0
                   + pv_rows[i, pl.ds(16, 16)] * u1
                   + pv_rows[i, pl.ds(32, 16)] * u2
                   + pv_rows[i, pl.ds(48, 16)] * u3)
            s = plsc.cumsum(acc)
            plsc.store_scatter(out_pos, [jnp.full((LANES,), i, jnp.int32)],
                               s, mask=m_last)

            def ng(n, carry3):
                r = i * N_NEG + n
                accn = (nv_rows[r, pl.ds(0, 16)] * u0
                        + nv_rows[r, pl.ds(16, 16)] * u1
                        + nv_rows[r, pl.ds(32, 16)] * u2
                        + nv_rows[r, pl.ds(48, 16)] * u3)
                sn = plsc.cumsum(accn)
                plsc.store_scatter(out_neg,
                                   [jnp.full((LANES,), r, jnp.int32)],
                                   sn, mask=m_last)
                return carry3

            lax.fori_loop(0, N_NEG, ng, 0)
            return carry2

        lax.fori_loop(0, C, ex, 0)
        pltpu.sync_copy(out_pos, pos_out_hbm.at[pl.ds(base, C)])
        pltpu.sync_copy(out_neg, neg_out_hbm.at[pl.ds(base * N_NEG, NEG_ROWS)])
        return carry

    lax.fori_loop(0, NCHUNK, chunk, 0)


def kernel(pos_u, pos_v, neg_v, U, V):
    pos_u_i = pos_u.astype(jnp.int32)
    pos_v_i = pos_v.astype(jnp.int32).reshape(B)
    neg_v_i = neg_v.astype(jnp.int32).reshape(B * N_NEG)

    mesh = plsc.VectorSubcoreMesh(core_axis_name="c", subcore_axis_name="s")
    run = pl.kernel(
        _sc_body,
        out_type=(jax.ShapeDtypeStruct((B,), jnp.float32),
                  jax.ShapeDtypeStruct((B * N_NEG,), jnp.float32)),
        mesh=mesh,
        compiler_params=pltpu.CompilerParams(
            needs_layout_passes=False, use_tc_tiling_on_sc=False),
        scratch_types=[
            pltpu.VMEM((C,), jnp.int32),
            pltpu.VMEM((C,), jnp.int32),
            pltpu.VMEM((NEG_ROWS,), jnp.int32),
            pltpu.VMEM((C, ROW_W), jnp.float32),
            pltpu.VMEM((C, ROW_W), jnp.float32),
            pltpu.VMEM((NEG_ROWS, ROW_W), jnp.float32),
            pltpu.VMEM((C,), jnp.float32),
            pltpu.VMEM((NEG_ROWS,), jnp.float32),
            pltpu.SemaphoreType.DMA,
            pltpu.SemaphoreType.DMA,
            pltpu.SemaphoreType.DMA,
        ],
    )
    pos_flat, neg_flat = run(pos_u_i, pos_v_i, neg_v_i, U, V)
    return (pos_flat.reshape(B, 1), neg_flat.reshape(B, N_NEG))


# speedup vs baseline: 5.3177x; 1.1294x over previous
import jax
import jax.numpy as jnp
from jax import lax
from jax.experimental import pallas as pl
from jax.experimental.pallas import tpu as pltpu
from jax.experimental.pallas import tpu_sc as plsc

B = 16384
D = 64
N_NEG = 20
LANES = 16
NW = 32
PER_W = B // NW
C = 16
NCHUNK = PER_W // C
NPAIR = NCHUNK // 2
NR = C * N_NEG
W128 = 128
VROWS = (1000000 * D) // W128


def _sc_body(pos_u_hbm, pos_v_hbm, neg_v_hbm, u_hbm, v_hbm,
             pos_out_hbm, neg_out_hbm,
             idx_u_st, idx_pv_st, idx_nv_st,
             t_u0, t_pv0, t_na0, t_nb0, t_nc0,
             t_u1, t_pv1, t_na1, t_nb1, t_nc1,
             u_r0, pv_r0, nv_r0, u_r1, pv_r1, nv_r1,
             out_pos, out_neg, sem0, sem1):
    wid = lax.axis_index("s") * 2 + lax.axis_index("c")
    lane = lax.iota(jnp.int32, LANES)

    pltpu.sync_copy(pos_u_hbm.at[pl.ds(wid * PER_W, PER_W)], idx_u_st)
    pltpu.sync_copy(pos_v_hbm.at[pl.ds(wid * PER_W, PER_W)], idx_pv_st)
    pltpu.sync_copy(neg_v_hbm.at[pl.ds(wid * PER_W * N_NEG, PER_W * N_NEG)],
                    idx_nv_st)

    def decode(c, t_u, t_pv, t_na, t_nb, t_nc):
        t_u[...] = idx_u_st[pl.ds(c * C, C)] >> 1
        t_pv[...] = idx_pv_st[pl.ds(c * C, C)] >> 1
        for j in range(8):
            t_na[pl.ds(j * LANES, LANES)] = (
                idx_nv_st[pl.ds(c * NR + j * LANES, LANES)] >> 1)
        for j in range(8):
            t_nb[pl.ds(j * LANES, LANES)] = (
                idx_nv_st[pl.ds(c * NR + (8 + j) * LANES, LANES)] >> 1)
        for j in range(4):
            t_nc[pl.ds(j * LANES, LANES)] = (
                idx_nv_st[pl.ds(c * NR + (16 + j) * LANES, LANES)] >> 1)

    def fire(t_u, t_pv, t_na, t_nb, t_nc, u_r, pv_r, nv_r, sem):
        pltpu.async_copy(u_hbm.at[t_u], u_r, sem)
        pltpu.async_copy(v_hbm.at[t_pv], pv_r, sem)
        pltpu.async_copy(v_hbm.at[t_na], nv_r.at[pl.ds(0, 128)], sem)
        pltpu.async_copy(v_hbm.at[t_nb], nv_r.at[pl.ds(128, 128)], sem)
        pltpu.async_copy(v_hbm.at[t_nc], nv_r.at[pl.ds(256, 64)], sem)

    def drain(u_r, pv_r, nv_r, sem):
        pltpu.make_async_copy(u_hbm.at[pl.ds(0, C)], u_r, sem).wait()
        pltpu.make_async_copy(v_hbm.at[pl.ds(0, C)], pv_r, sem).wait()
        pltpu.make_async_copy(v_hbm.at[pl.ds(0, NR)], nv_r, sem).wait()

    def dots(u_r, hu, refs, rows, hcols):
        k = len(refs)

        def step(d, accs):
            rot = (lane + d) & (D - 1)
            gu = plsc.load_gather(u_r, [lane, hu + rot])
            out = []
            for j in range(k):
                g = plsc.load_gather(refs[j], [rows[j], hcols[j] + rot])
                out.append(accs[j] + gu * g)
            return tuple(out)

        zero = jnp.zeros((LANES,), jnp.float32)
        return lax.fori_loop(0, D, step, tuple([zero] * k))

    def compute(c, u_r, pv_r, nv_r):
        hu = (idx_u_st[pl.ds(c * C, C)] & 1) << 6
        hpv = (idx_pv_st[pl.ds(c * C, C)] & 1) << 6
        hn = []
        for n in range(N_NEG):
            iv = plsc.load_gather(idx_nv_st, [lane * N_NEG + (c * NR + n)])
            hn.append((iv & 1) << 6)
        nrows = [lane * N_NEG + n for n in range(N_NEG)]

        accs_a = dots(u_r, hu,
                      [pv_r] + [nv_r] * 10,
                      [lane] + nrows[:10],
                      [hpv] + hn[:10])
        out_pos[pl.ds(c * C, C)] = accs_a[0]
        for n in range(10):
            out_neg[pl.ds(n * PER_W + c * C, C)] = accs_a[1 + n]

        accs_b = dots(u_r, hu, [nv_r] * 10, nrows[10:], hn[10:])
        for n in range(10):
            out_neg[pl.ds((10 + n) * PER_W + c * C, C)] = accs_b[n]

    decode(0, t_u0, t_pv0, t_na0, t_nb0, t_nc0)
    fire(t_u0, t_pv0, t_na0, t_nb0, t_nc0, u_r0, pv_r0, nv_r0, sem0)

    def pair(i, carry):
        c = 2 * i
        decode(c + 1, t_u1, t_pv1, t_na1, t_nb1, t_nc1)
        fire(t_u1, t_pv1, t_na1, t_nb1, t_nc1, u_r1, pv_r1, nv_r1, sem1)
        drain(u_r0, pv_r0, nv_r0, sem0)
        compute(c, u_r0, pv_r0, nv_r0)

        @pl.when(i < NPAIR - 1)
        def _():
            decode(c + 2, t_u0, t_pv0, t_na0, t_nb0, t_nc0)
            fire(t_u0, t_pv0, t_na0, t_nb0, t_nc0, u_r0, pv_r0, nv_r0, sem0)

        drain(u_r1, pv_r1, nv_r1, sem1)
        compute(c + 1, u_r1, pv_r1, nv_r1)
        return carry

    lax.fori_loop(0, NPAIR, pair, 0)

    pltpu.sync_copy(out_pos, pos_out_hbm.at[pl.ds(wid * PER_W, PER_W)])
    for n in range(N_NEG):
        pltpu.sync_copy(out_neg.at[pl.ds(n * PER_W, PER_W)],
                        neg_out_hbm.at[pl.ds(n * B + wid * PER_W, PER_W)])


def kernel(pos_u, pos_v, neg_v, U, V):
    pos_u_i = pos_u.astype(jnp.int32)
    pos_v_i = pos_v.astype(jnp.int32).reshape(B)
    neg_v_i = neg_v.astype(jnp.int32).reshape(B * N_NEG)
    u2 = U.reshape(VROWS, W128)
    v2 = V.reshape(VROWS, W128)

    mesh = plsc.VectorSubcoreMesh(core_axis_name="c", subcore_axis_name="s")
    run = pl.kernel(
        _sc_body,
        out_type=(jax.ShapeDtypeStruct((B,), jnp.float32),
                  jax.ShapeDtypeStruct((N_NEG * B,), jnp.float32)),
        mesh=mesh,
        compiler_params=pltpu.CompilerParams(needs_layout_passes=False),
        scratch_types=[
            pltpu.VMEM((PER_W,), jnp.int32),
            pltpu.VMEM((PER_W,), jnp.int32),
            pltpu.VMEM((PER_W * N_NEG,), jnp.int32),
            pltpu.VMEM((C,), jnp.int32),
            pltpu.VMEM((C,), jnp.int32),
            pltpu.VMEM((128,), jnp.int32),
            pltpu.VMEM((128,), jnp.int32),
            pltpu.VMEM((64,), jnp.int32),
            pltpu.VMEM((C,), jnp.int32),
            pltpu.VMEM((C,), jnp.int32),
            pltpu.VMEM((128,), jnp.int32),
            pltpu.VMEM((128,), jnp.int32),
            pltpu.VMEM((64,), jnp.int32),
            pltpu.VMEM((C, W128), jnp.float32),
            pltpu.VMEM((C, W128), jnp.float32),
            pltpu.VMEM((NR, W128), jnp.float32),
            pltpu.VMEM((C, W128), jnp.float32),
            pltpu.VMEM((C, W128), jnp.float32),
            pltpu.VMEM((NR, W128), jnp.float32),
            pltpu.VMEM((PER_W,), jnp.float32),
            pltpu.VMEM((PER_W * N_NEG,), jnp.float32),
            pltpu.SemaphoreType.DMA,
            pltpu.SemaphoreType.DMA,
        ],
    )
    pos_flat, neg_flat = run(pos_u_i, pos_v_i, neg_v_i, u2, v2)
    return (pos_flat.reshape(B, 1),
            neg_flat.reshape(N_NEG, B).T)
